# Initial kernel scaffold; baseline (speedup 1.0000x reference)
import jax
import jax.numpy as jnp
from jax.experimental import pallas as pl


def kernel(ui_edge_index, uu_edge_index, pos_edge_index, neg_edge_index, params):
    raise NotImplementedError("write your pallas kernel here")


# speedup vs baseline: 3.9565x; 3.9565x over previous
import functools

import jax
import jax.numpy as jnp
from jax import lax
from jax.experimental import pallas as pl
from jax.experimental.pallas import tpu as pltpu
from jax.experimental.pallas import tpu_sc as plsc

N = 50000
D = 64
DH = 32
E = 800000
EP = 100000
B = 128
NC = 2
NS = 16
NW = NC * NS
NBR = E // B
KB1 = 200
NBT = KB1 * NW
EPAD = NBT * B
KB2 = NBT // NS
CH2 = 50
FC = 1000
NFC = N // FC

PB = 80
KBP = 80
PNBT = KBP * NW
PPAD = PNBT * PB
RD = 3 * D

_mesh = plsc.VectorSubcoreMesh(core_axis_name="c", subcore_axis_name="s")


_R = 2000


def _proj_body(u_ref, i_ref,
               wsr_ref, bsr_ref, wdr_ref, bdr_ref,
               wsb_ref, bsb_ref, wdb_ref, bdb_ref,
               wss_ref, bss_ref, wds_ref, bds_ref,
               el_r_ref, er_r_ref, el_b_ref, er_b_ref, el_s_ref, er_s_ref):
    u = u_ref[...]
    it = i_ref[...]

    def proj(x, w_ref, b_ref):
        y = lax.dot_general(x, w_ref[...], (((1,), (0,)), ((), ())),
                            preferred_element_type=jnp.float32)
        return y + b_ref[...]

    def split_store(y, o_ref):
        o_ref[0, :, :] = y[:, :DH]
        o_ref[1, :, :] = y[:, DH:]

    split_store(proj(u, wsr_ref, bsr_ref), el_r_ref)
    split_store(proj(it, wdr_ref, bdr_ref), er_r_ref)
    split_store(proj(it, wsb_ref, bsb_ref), el_b_ref)
    split_store(proj(u, wdb_ref, bdb_ref), er_b_ref)
    split_store(proj(u, wss_ref, bss_ref), el_s_ref)
    split_store(proj(u, wds_ref, bds_ref), er_s_ref)


def _proj(emb_u, emb_i, lp):
    n_blocks = N // _R
    emb_spec = pl.BlockSpec((_R, D), lambda i: (i, 0))
    w_spec = pl.BlockSpec((D, D), lambda i: (0, 0))
    b_spec = pl.BlockSpec((1, D), lambda i: (0, 0))
    o_spec = pl.BlockSpec((2, _R, DH), lambda i: (0, i, 0))
    out_shape = jax.ShapeDtypeStruct((2, N, DH), jnp.float32)
    return pl.pallas_call(
        _proj_body,
        grid=(n_blocks,),
        in_specs=[emb_spec, emb_spec] + [w_spec, b_spec] * 6,
        out_specs=[o_spec] * 6,
        out_shape=[out_shape] * 6,
    )(emb_u, emb_i,
      lp["rate"]["Wsrc"], lp["rate"]["bsrc"].reshape(1, D),
      lp["rate"]["Wdst"], lp["rate"]["bdst"].reshape(1, D),
      lp["rated_by"]["Wsrc"], lp["rated_by"]["bsrc"].reshape(1, D),
      lp["rated_by"]["Wdst"], lp["rated_by"]["bdst"].reshape(1, D),
      lp["social"]["Wsrc"], lp["social"]["bsrc"].reshape(1, D),
      lp["social"]["Wdst"], lp["social"]["bdst"].reshape(1, D))


def _k1_body(ell_ref, elh_ref, erl_ref, erh_ref, attn_ref,
             src_ref, dst_ref, zer_ref, ex_ref, den_ref,
             idx_s, idx_d,
             ell0, elh0, erl0, erh0, ell1, elh1, erl1, erh1,
             exb0, exb1, ex8_v, attn_vm, attn_sh, attn_sm, den_sh,
             sem0, sem1, osem0, osem1):
    ci = lax.axis_index("c")
    sid = lax.axis_index("s")
    w = sid * NC + ci

    def zero_body(i, _):
        k = i * NS + sid
        pltpu.sync_copy(zer_ref, den_sh.at[pl.ds(k * FC, FC)])
        return 0

    nzc = NFC // NS + jnp.where(sid < NFC % NS, 1, 0)
    lax.fori_loop(0, nzc, zero_body, 0)
    pltpu.sync_copy(zer_ref.at[pl.ds(0, B)], ex8_v)
    pltpu.sync_copy(attn_ref, attn_vm)
    pltpu.sync_copy(attn_vm, attn_sh)
    pltpu.sync_copy(attn_sh, attn_sm)
    plsc.subcore_barrier()

    pltpu.sync_copy(src_ref.at[pl.ds(w * KB1, KB1)], idx_s)
    pltpu.sync_copy(dst_ref.at[pl.ds(w * KB1, KB1)], idx_d)

    bufs = [(ell0, elh0, erl0, erh0, sem0, exb0, osem0),
            (ell1, elh1, erl1, erh1, sem1, exb1, osem1)]
    iota = jax.lax.iota(jnp.int32, 16)

    def fire(i, buf):
        el_lo, el_hi, er_lo, er_hi, sem = buf[:5]
        pltpu.async_copy(ell_ref.at[idx_s.at[i]], el_lo, sem)
        pltpu.async_copy(elh_ref.at[idx_s.at[i]], el_hi, sem)
        pltpu.async_copy(erl_ref.at[idx_d.at[i]], er_lo, sem)
        pltpu.async_copy(erh_ref.at[idx_d.at[i]], er_hi, sem)

    def drain(buf):
        el_lo, el_hi, er_lo, er_hi, sem = buf[:5]
        pltpu.make_async_copy(ell_ref.at[pl.ds(0, B)], el_lo, sem).wait()
        pltpu.make_async_copy(elh_ref.at[pl.ds(0, B)], el_hi, sem).wait()
        pltpu.make_async_copy(erl_ref.at[pl.ds(0, B)], er_lo, sem).wait()
        pltpu.make_async_copy(erh_ref.at[pl.ds(0, B)], er_hi, sem).wait()

    def compute(i, buf):
        el_lo, el_hi, er_lo, er_hi, _, exb, osem = buf
        bi = w * KB1 + i
        m = jnp.where(bi < NBR, jnp.float32(1.0), jnp.float32(0.0))

        @pl.when(i >= 2)
        def _():
            pltpu.make_async_copy(exb, ex_ref.at[pl.ds(0, B)], osem).wait()

        def group_body(g, _):
            ridx = g * 16 + iota

            def dim_body(d, score):
                cidx = jnp.broadcast_to(d, (16,))
                xlo = (plsc.load_gather(el_lo, [ridx, cidx]) +
                       plsc.load_gather(er_lo, [ridx, cidx]))
                xlo = jnp.maximum(xlo, 0.2 * xlo)
                xhi = (plsc.load_gather(el_hi, [ridx, cidx]) +
                       plsc.load_gather(er_hi, [ridx, cidx]))
                xhi = jnp.maximum(xhi, 0.2 * xhi)
                return score + attn_sm[d] * xlo + attn_sm[d + DH] * xhi

            score = lax.fori_loop(0, DH, dim_body,
                                  jnp.zeros((16,), jnp.float32))
            ex16 = jnp.exp(score) * m
            exb[pl.ds(g * 16, 16)] = ex16
            plsc.store_scatter(ex8_v, [ridx, jnp.zeros((16,), jnp.int32)],
                               ex16)
            return 0

        lax.fori_loop(0, B // 16, group_body, 0)
        pltpu.sync_copy(ex8_v, den_sh.at[idx_d.at[i]], add=True)
        pltpu.async_copy(exb, ex_ref.at[pl.ds(bi * B, B)], osem)

    fire(0, bufs[0])

    def body(g, _):
        i0 = 2 * g
        i1 = 2 * g + 1
        fire(i1, bufs[1])
        drain(bufs[0])
        compute(i0, bufs[0])

        @pl.when(i0 + 2 < KB1)
        def _():
            fire(i0 + 2, bufs[0])

        drain(bufs[1])
        compute(i1, bufs[1])

        @pl.when(i1 + 2 < KB1)
        def _():
            fire(i1 + 2, bufs[1])

        return 0

    lax.fori_loop(0, KB1 // 2, body, 0)
    pltpu.make_async_copy(exb0, ex_ref.at[pl.ds(0, B)], osem0).wait()
    pltpu.make_async_copy(exb1, ex_ref.at[pl.ds(0, B)], osem1).wait()
    plsc.subcore_barrier()

    def flush_body(i, _):
        k = i * NS + sid
        pltpu.sync_copy(den_sh.at[pl.ds(k * FC, FC)],
                        den_ref.at[ci, pl.ds(k * FC, FC)])
        return 0

    lax.fori_loop(0, nzc, flush_body, 0)


@functools.partial(
    pl.kernel,
    mesh=_mesh,
    compiler_params=pltpu.CompilerParams(needs_layout_passes=False,
                                         use_tc_tiling_on_sc=False),
    out_type=[jax.ShapeDtypeStruct((EPAD,), jnp.float32),
              jax.ShapeDtypeStruct((NC, N, 8), jnp.float32)],
    scratch_types=[
        pltpu.VMEM((KB1, B), jnp.int32),
        pltpu.VMEM((KB1, B), jnp.int32),
        pltpu.VMEM((B, DH), jnp.float32),
        pltpu.VMEM((B, DH), jnp.float32),
        pltpu.VMEM((B, DH), jnp.float32),
        pltpu.VMEM((B, DH), jnp.float32),
        pltpu.VMEM((B, DH), jnp.float32),
        pltpu.VMEM((B, DH), jnp.float32),
        pltpu.VMEM((B, DH), jnp.float32),
        pltpu.VMEM((B, DH), jnp.float32),
        pltpu.VMEM((B,), jnp.float32),
        pltpu.VMEM((B,), jnp.float32),
        pltpu.VMEM((B, 8), jnp.float32),
        pltpu.VMEM((D,), jnp.float32),
        pltpu.VMEM_SHARED((D,), jnp.float32),
        pltpu.SMEM((D,), jnp.float32),
        pltpu.VMEM_SHARED((N, 8), jnp.float32),
        pltpu.SemaphoreType.DMA,
        pltpu.SemaphoreType.DMA,
        pltpu.SemaphoreType.DMA,
        pltpu.SemaphoreType.DMA,
    ],
)
def _k1(ell_ref, elh_ref, erl_ref, erh_ref, attn_ref, src_ref, dst_ref,
        zer_ref, ex_ref, den_ref, *scratch):
    _k1_body(ell_ref, elh_ref, erl_ref, erh_ref, attn_ref,
             src_ref, dst_ref, zer_ref, ex_ref, den_ref, *scratch)


def _k2_body(ell_ref, elh_ref, ex_ref, src_ref, dst_ref, zer_ref, out_ref,
             idx_s, idx_d, rows0, exv0, rows1, exv1, out_sh, sem0, sem1):
    ci = lax.axis_index("c")
    sid = lax.axis_index("s")

    def zero_body(i, _):
        k = i * NS + sid
        pltpu.sync_copy(zer_ref, out_sh.at[pl.ds(k * FC, FC)])
        return 0

    nzc = NFC // NS + jnp.where(sid < NFC % NS, 1, 0)
    lax.fori_loop(0, nzc, zero_body, 0)
    plsc.subcore_barrier()

    bufs = [(rows0, exv0, sem0), (rows1, exv1, sem1)]
    iota = jax.lax.iota(jnp.int32, 16)

    def fire(c, i, buf):
        rows_v, ex_v, sem = buf
        bi = sid * KB2 + c * CH2 + i

        @pl.when(ci == 0)
        def _():
            pltpu.async_copy(ell_ref.at[idx_s.at[i]], rows_v, sem)

        @pl.when(ci == 1)
        def _():
            pltpu.async_copy(elh_ref.at[idx_s.at[i]], rows_v, sem)

        pltpu.async_copy(ex_ref.at[pl.ds(bi * B, B)], ex_v, sem)

    def drain(buf):
        rows_v, ex_v, sem = buf
        pltpu.make_async_copy(ell_ref.at[pl.ds(0, B)], rows_v, sem).wait()
        pltpu.make_async_copy(ex_ref.at[pl.ds(0, B)], ex_v, sem).wait()

    def compute(i, buf):
        rows_v, ex_v, _ = buf

        def group_body(g, _):
            ridx = g * 16 + iota
            ex16 = ex_v[pl.ds(g * 16, 16)]

            def dim_body(d, _):
                cidx = jnp.broadcast_to(d, (16,))
                col = plsc.load_gather(rows_v, [ridx, cidx])
                plsc.store_scatter(rows_v, [ridx, cidx], col * ex16)
                return 0

            lax.fori_loop(0, DH, dim_body, 0)
            return 0

        lax.fori_loop(0, B // 16, group_body, 0)
        pltpu.sync_copy(rows_v, out_sh.at[idx_d.at[i]], add=True)

    def chunk_body(c, _):
        pltpu.sync_copy(src_ref.at[pl.ds(sid * KB2 + c * CH2, CH2)], idx_s)
        pltpu.sync_copy(dst_ref.at[pl.ds(sid * KB2 + c * CH2, CH2)], idx_d)
        fire(c, 0, bufs[0])

        def body(g, _):
            i0 = 2 * g
            i1 = 2 * g + 1
            fire(c, i1, bufs[1])
            drain(bufs[0])
            compute(i0, bufs[0])

            @pl.when(i0 + 2 < CH2)
            def _():
                fire(c, i0 + 2, bufs[0])

            drain(bufs[1])
            compute(i1, bufs[1])

            @pl.when(i1 + 2 < CH2)
            def _():
                fire(c, i1 + 2, bufs[1])

            return 0

        lax.fori_loop(0, CH2 // 2, body, 0)
        return 0

    lax.fori_loop(0, KB2 // CH2, chunk_body, 0)
    plsc.subcore_barrier()

    def flush_body(i, _):
        k = i * NS + sid
        pltpu.sync_copy(out_sh.at[pl.ds(k * FC, FC)],
                        out_ref.at[ci, pl.ds(k * FC, FC)])
        return 0

    lax.fori_loop(0, nzc, flush_body, 0)


@functools.partial(
    pl.kernel,
    mesh=_mesh,
    compiler_params=pltpu.CompilerParams(needs_layout_passes=False,
                                         use_tc_tiling_on_sc=False),
    out_type=jax.ShapeDtypeStruct((NC, N, DH), jnp.float32),
    scratch_types=[
        pltpu.VMEM((CH2, B), jnp.int32),
        pltpu.VMEM((CH2, B), jnp.int32),
        pltpu.VMEM((B, DH), jnp.float32),
        pltpu.VMEM((B,), jnp.float32),
        pltpu.VMEM((B, DH), jnp.float32),
        pltpu.VMEM((B,), jnp.float32),
        pltpu.VMEM_SHARED((N, DH), jnp.float32),
        pltpu.SemaphoreType.DMA,
        pltpu.SemaphoreType.DMA,
    ],
)
def _k2(ell_ref, elh_ref, ex_ref, src_ref, dst_ref, zer_ref, out_ref,
        *scratch):
    _k2_body(ell_ref, elh_ref, ex_ref, src_ref, dst_ref, zer_ref, out_ref,
             *scratch)


def _a1_body(u_ref, i_ref, outr_ref, outb_ref, outs_ref,
             denr_ref, denb_ref, dens_ref,
             w1f_ref, b1f_ref, w2f_ref, b2f_ref,
             w1t_ref, b1t_ref, w2t_ref, b2t_ref,
             inew_ref, p_ref, q_ref, sinf_ref, sint_ref, mom_ref):
    step = pl.program_id(0)

    def seg_norm(o_ref, den_ref):
        den = den_ref[0, :, 0] + den_ref[1, :, 0] + 1e-9
        num = jnp.concatenate([o_ref[0, :, :], o_ref[1, :, :]], axis=1)
        return num / den[:, None]

    item_agg = seg_norm(outr_ref, denr_ref)
    inew_ref[...] = item_agg + i_ref[...]
    q = seg_norm(outb_ref, denb_ref)
    p = seg_norm(outs_ref, dens_ref)
    p_ref[...] = p
    q_ref[...] = q

    u = u_ref[...]

    def att_s(xr, w1_ref, b1_ref, w2_ref, b2_ref):
        x = jnp.concatenate([u, xr], axis=1)
        h = lax.dot_general(x, w1_ref[...], (((1,), (0,)), ((), ())),
                            preferred_element_type=jnp.float32) + b1_ref[...]
        s = lax.dot_general(h, w2_ref[...], (((1,), (0,)), ((), ())),
                            preferred_element_type=jnp.float32) + b2_ref[...]
        return s

    s_inf = att_s(p, w1f_ref, b1f_ref, w2f_ref, b2f_ref)
    s_int = att_s(q, w1t_ref, b1t_ref, w2t_ref, b2t_ref)
    sinf_ref[...] = s_inf
    sint_ref[...] = s_int

    @pl.when(step == 0)
    def _():
        mom_ref[...] = jnp.zeros((1, 8), jnp.float32)

    upd = jnp.concatenate([
        jnp.sum(s_inf)[None], jnp.sum(s_inf * s_inf)[None],
        jnp.sum(s_int)[None], jnp.sum(s_int * s_int)[None],
        jnp.zeros((4,), jnp.float32)]).reshape(1, 8)
    mom_ref[...] = mom_ref[...] + upd


def _a1(emb_u, emb_i, outr, outb, outs, denr, denb, dens, att_inf, att_int):
    n_blocks = N // _R
    emb_spec = pl.BlockSpec((_R, D), lambda i: (i, 0))
    o_spec = pl.BlockSpec((2, _R, DH), lambda i: (0, i, 0))
    den_spec = pl.BlockSpec((2, _R, 8), lambda i: (0, i, 0))
    w1_spec = pl.BlockSpec((2 * D, 2 * D), lambda i: (0, 0))
    b1_spec = pl.BlockSpec((1, 2 * D), lambda i: (0, 0))
    w2_spec = pl.BlockSpec((2 * D, 1), lambda i: (0, 0))
    b2_spec = pl.BlockSpec((1, 1), lambda i: (0, 0))
    s_spec = pl.BlockSpec((_R, 1), lambda i: (i, 0))
    mom_spec = pl.BlockSpec((1, 8), lambda i: (0, 0))
    return pl.pallas_call(
        _a1_body,
        grid=(n_blocks,),
        in_specs=[emb_spec, emb_spec, o_spec, o_spec, o_spec,
                  den_spec, den_spec, den_spec,
                  w1_spec, b1_spec, w2_spec, b2_spec,
                  w1_spec, b1_spec, w2_spec, b2_spec],
        out_specs=[emb_spec, emb_spec, emb_spec, s_spec, s_spec, mom_spec],
        out_shape=[jax.ShapeDtypeStruct((N, D), jnp.float32),
                   jax.ShapeDtypeStruct((N, D), jnp.float32),
                   jax.ShapeDtypeStruct((N, D), jnp.float32),
                   jax.ShapeDtypeStruct((N, 1), jnp.float32),
                   jax.ShapeDtypeStruct((N, 1), jnp.float32),
                   jax.ShapeDtypeStruct((1, 8), jnp.float32)],
    )(emb_u, emb_i, outr, outb, outs, denr, denb, dens,
      att_inf["W1"], att_inf["b1"].reshape(1, 2 * D),
      att_inf["W2"], att_inf["b2"].reshape(1, 1),
      att_int["W1"], att_int["b1"].reshape(1, 2 * D),
      att_int["W2"], att_int["b2"].reshape(1, 1))


def _a2_body(p_ref, q_ref, u_ref, sinf_ref, sint_ref, mom_ref, unew_ref):
    n_f = jnp.float32(N)
    mu_f = mom_ref[0, 0] / n_f
    var_f = mom_ref[0, 1] / n_f - mu_f * mu_f
    mu_t = mom_ref[0, 2] / n_f
    var_t = mom_ref[0, 3] / n_f - mu_t * mu_t
    a = (sinf_ref[...] - mu_f) * lax.rsqrt(var_f + 1e-5)
    b = (sint_ref[...] - mu_t) * lax.rsqrt(var_t + 1e-5)
    a = jnp.maximum(a, 0.01 * a)
    b = jnp.maximum(b, 0.01 * b)
    m = jnp.maximum(a, b)
    ea = jnp.exp(a - m)
    eb = jnp.exp(b - m)
    tot = ea + eb
    g0 = ea / tot
    g1 = eb / tot
    unew_ref[...] = g0 * p_ref[...] + g1 * q_ref[...] + u_ref[...]


def _a2(p_hair, q_hair, emb_u, s_inf, s_int, mom):
    n_blocks = N // _R
    emb_spec = pl.BlockSpec((_R, D), lambda i: (i, 0))
    s_spec = pl.BlockSpec((_R, 1), lambda i: (i, 0))
    mom_spec = pl.BlockSpec((1, 8), lambda i: (0, 0))
    return pl.pallas_call(
        _a2_body,
        grid=(n_blocks,),
        in_specs=[emb_spec, emb_spec, emb_spec, s_spec, s_spec, mom_spec],
        out_specs=emb_spec,
        out_shape=jax.ShapeDtypeStruct((N, D), jnp.float32),
    )(p_hair, q_hair, emb_u, s_inf, s_int, mom)


def _pred_body(ru_ref, ri_ref, iu_ref, ii_ref, out_ref,
               idx_u, idx_i, ru0, ri0, ru1, ri1, sc_v, sem0, sem1):
    ci = lax.axis_index("c")
    sid = lax.axis_index("s")
    w = sid * NC + ci
    iota = jax.lax.iota(jnp.int32, 16)

    pltpu.sync_copy(iu_ref.at[pl.ds(w * KBP, KBP)], idx_u)
    pltpu.sync_copy(ii_ref.at[pl.ds(w * KBP, KBP)], idx_i)

    bufs = [(ru0, ri0, sem0), (ru1, ri1, sem1)]

    def fire(i, buf):
        ru_v, ri_v, sem = buf
        pltpu.async_copy(ru_ref.at[idx_u.at[i]], ru_v, sem)
        pltpu.async_copy(ri_ref.at[idx_i.at[i]], ri_v, sem)

    def drain(buf):
        ru_v, ri_v, sem = buf
        pltpu.make_async_copy(ru_ref.at[pl.ds(0, PB)], ru_v, sem).wait()
        pltpu.make_async_copy(ri_ref.at[pl.ds(0, PB)], ri_v, sem).wait()

    def compute(i, buf):
        ru_v, ri_v, _ = buf

        def group_body(g, _):
            ridx = g * 16 + iota

            def dim_body(d, acc):
                cidx = jnp.broadcast_to(d, (16,))
                uu = plsc.load_gather(ru_v, [ridx, cidx])
                vv = plsc.load_gather(ri_v, [ridx, cidx])
                return acc + uu * vv

            acc = lax.fori_loop(0, RD, dim_body, jnp.zeros((16,), jnp.float32))
            sc_v[pl.ds(i * PB + g * 16, 16)] = acc
            return 0

        lax.fori_loop(0, PB // 16, group_body, 0)

    fire(0, bufs[0])

    def body(g, _):
        i0 = 2 * g
        i1 = 2 * g + 1
        fire(i1, bufs[1])
        drain(bufs[0])
        compute(i0, bufs[0])

        @pl.when(i0 + 2 < KBP)
        def _():
            fire(i0 + 2, bufs[0])

        drain(bufs[1])
        compute(i1, bufs[1])

        @pl.when(i1 + 2 < KBP)
        def _():
            fire(i1 + 2, bufs[1])

        return 0

    lax.fori_loop(0, KBP // 2, body, 0)
    pltpu.sync_copy(sc_v, out_ref.at[pl.ds(w * KBP * PB, KBP * PB)])


@functools.partial(
    pl.kernel,
    mesh=_mesh,
    compiler_params=pltpu.CompilerParams(needs_layout_passes=False,
                                         use_tc_tiling_on_sc=False),
    out_type=jax.ShapeDtypeStruct((PPAD,), jnp.float32),
    scratch_types=[
        pltpu.VMEM((KBP, PB), jnp.int32),
        pltpu.VMEM((KBP, PB), jnp.int32),
        pltpu.VMEM((PB, RD), jnp.float32),
        pltpu.VMEM((PB, RD), jnp.float32),
        pltpu.VMEM((PB, RD), jnp.float32),
        pltpu.VMEM((PB, RD), jnp.float32),
        pltpu.VMEM((KBP * PB,), jnp.float32),
        pltpu.SemaphoreType.DMA,
        pltpu.SemaphoreType.DMA,
    ],
)
def _pred(ru_ref, ri_ref, iu_ref, ii_ref, *scratch):
    _pred_body(ru_ref, ri_ref, iu_ref, ii_ref, *scratch)


def _pad2d(x, nb, b):
    x = x.astype(jnp.int32)
    pad = nb * b - x.shape[0]
    return jnp.concatenate([x, jnp.zeros((pad,), jnp.int32)]).reshape(nb, b)


def kernel(ui_edge_index, uu_edge_index, pos_edge_index, neg_edge_index, params):
    ui0 = _pad2d(ui_edge_index[0], NBT, B)
    ui1 = _pad2d(ui_edge_index[1], NBT, B)
    uu0 = _pad2d(uu_edge_index[0], NBT, B)
    uu1 = _pad2d(uu_edge_index[1], NBT, B)

    zer8 = jnp.zeros((FC, 8), jnp.float32)
    zer32 = jnp.zeros((FC, DH), jnp.float32)

    emb_u = params["user_emb"]
    emb_i = params["item_emb"]
    res_u = [emb_u]
    res_i = [emb_i]

    for lp in params["layers"]:
        el_r, er_r, el_b, er_b, el_s, er_s = _proj(emb_u, emb_i, lp)

        ex_r, den_r = _k1(el_r[0], el_r[1], er_r[0], er_r[1],
                          lp["rate"]["attn"], ui0, ui1, zer8)
        ex_b, den_b = _k1(el_b[0], el_b[1], er_b[0], er_b[1],
                          lp["rated_by"]["attn"], ui1, ui0, zer8)
        ex_s, den_s = _k1(el_s[0], el_s[1], er_s[0], er_s[1],
                          lp["social"]["attn"], uu0, uu1, zer8)

        outr = _k2(el_r[0], el_r[1], ex_r, ui0, ui1, zer32)
        outb = _k2(el_b[0], el_b[1], ex_b, ui1, ui0, zer32)
        outs = _k2(el_s[0], el_s[1], ex_s, uu0, uu1, zer32)

        emb_i_new, p_hair, q_hair, s_inf, s_int, mom = _a1(
            emb_u, emb_i, outr, outb, outs, den_r, den_b, den_s,
            lp["att_inf"], lp["att_int"])
        emb_u = _a2(p_hair, q_hair, emb_u, s_inf, s_int, mom)
        emb_i = emb_i_new
        res_u.append(emb_u)
        res_i.append(emb_i)

    res_u_cat = jnp.concatenate(res_u, axis=1)
    res_i_cat = jnp.concatenate(res_i, axis=1)

    iu = _pad2d(jnp.concatenate([pos_edge_index[0], neg_edge_index[0]]),
                PNBT, PB)
    ii = _pad2d(jnp.concatenate([pos_edge_index[1], neg_edge_index[1]]),
                PNBT, PB)
    scores = _pred(res_u_cat, res_i_cat, iu, ii)
    pos_score = scores[:EP, None]
    neg_score = scores[EP:2 * EP, None]
    return pos_score, neg_score


# speedup vs baseline: 4.1254x; 1.0427x over previous
import functools

import jax
import jax.numpy as jnp
from jax import lax
from jax.experimental import pallas as pl
from jax.experimental.pallas import tpu as pltpu
from jax.experimental.pallas import tpu_sc as plsc

N = 50000
D = 64
DH = 32
E = 800000
EP = 100000
B = 128
NC = 2
NS = 16
NW = NC * NS
NBR = E // B
KB1 = 200
NBT = KB1 * NW
EPAD = NBT * B
KB2 = NBT // NS
CH2 = 50
FC = 1000
NFC = N // FC

PB = 80
KBP = 80
PNBT = KBP * NW
PPAD = PNBT * PB
RD = 3 * D

_mesh = plsc.VectorSubcoreMesh(core_axis_name="c", subcore_axis_name="s")


_R = 2000


def _proj_body(u_ref, i_ref,
               wsr_ref, bsr_ref, wdr_ref, bdr_ref,
               wsb_ref, bsb_ref, wdb_ref, bdb_ref,
               wss_ref, bss_ref, wds_ref, bds_ref,
               el_r_ref, el64_r_ref, er64_r_ref,
               el_b_ref, el64_b_ref, er64_b_ref,
               el_s_ref, el64_s_ref, er64_s_ref):
    u = u_ref[...]
    it = i_ref[...]

    def proj(x, w_ref, b_ref):
        y = lax.dot_general(x, w_ref[...], (((1,), (0,)), ((), ())),
                            preferred_element_type=jnp.float32)
        return y + b_ref[...]

    def split_store(y, o_ref):
        o_ref[0, :, :] = y[:, :DH]
        o_ref[1, :, :] = y[:, DH:]

    def el_store(y, split_ref, full_ref):
        split_store(y, split_ref)
        full_ref[...] = y

    el_store(proj(u, wsr_ref, bsr_ref), el_r_ref, el64_r_ref)
    er64_r_ref[...] = proj(it, wdr_ref, bdr_ref)
    el_store(proj(it, wsb_ref, bsb_ref), el_b_ref, el64_b_ref)
    er64_b_ref[...] = proj(u, wdb_ref, bdb_ref)
    el_store(proj(u, wss_ref, bss_ref), el_s_ref, el64_s_ref)
    er64_s_ref[...] = proj(u, wds_ref, bds_ref)


def _proj(emb_u, emb_i, lp):
    n_blocks = N // _R
    emb_spec = pl.BlockSpec((_R, D), lambda i: (i, 0))
    w_spec = pl.BlockSpec((D, D), lambda i: (0, 0))
    b_spec = pl.BlockSpec((1, D), lambda i: (0, 0))
    o_spec = pl.BlockSpec((2, _R, DH), lambda i: (0, i, 0))
    out_shape = jax.ShapeDtypeStruct((2, N, DH), jnp.float32)
    out_shape64 = jax.ShapeDtypeStruct((N, D), jnp.float32)
    return pl.pallas_call(
        _proj_body,
        grid=(n_blocks,),
        in_specs=[emb_spec, emb_spec] + [w_spec, b_spec] * 6,
        out_specs=[o_spec, emb_spec, emb_spec] * 3,
        out_shape=[out_shape, out_shape64, out_shape64] * 3,
    )(emb_u, emb_i,
      lp["rate"]["Wsrc"], lp["rate"]["bsrc"].reshape(1, D),
      lp["rate"]["Wdst"], lp["rate"]["bdst"].reshape(1, D),
      lp["rated_by"]["Wsrc"], lp["rated_by"]["bsrc"].reshape(1, D),
      lp["rated_by"]["Wdst"], lp["rated_by"]["bdst"].reshape(1, D),
      lp["social"]["Wsrc"], lp["social"]["bsrc"].reshape(1, D),
      lp["social"]["Wdst"], lp["social"]["bdst"].reshape(1, D))


def _k1_body(el_ref, er_ref, attn_ref,
             src_ref, dst_ref, zer_ref, ex_ref, den_ref,
             idx_s, idx_d,
             el0, er0, el1, er1,
             exb0, exb1, ex8_v, attn_vm, attn_sh, attn_sm, den_sh,
             sem0, sem1, osem0, osem1):
    ci = lax.axis_index("c")
    sid = lax.axis_index("s")
    w = sid * NC + ci

    def zero_body(i, _):
        k = i * NS + sid
        pltpu.sync_copy(zer_ref, den_sh.at[pl.ds(k * FC, FC)])
        return 0

    nzc = NFC // NS + jnp.where(sid < NFC % NS, 1, 0)
    lax.fori_loop(0, nzc, zero_body, 0)
    pltpu.sync_copy(zer_ref.at[pl.ds(0, B)], ex8_v)
    pltpu.sync_copy(attn_ref, attn_vm)
    pltpu.sync_copy(attn_vm, attn_sh)
    pltpu.sync_copy(attn_sh, attn_sm)
    plsc.subcore_barrier()

    pltpu.sync_copy(src_ref.at[pl.ds(w * KB1, KB1)], idx_s)
    pltpu.sync_copy(dst_ref.at[pl.ds(w * KB1, KB1)], idx_d)

    bufs = [(el0, er0, sem0, exb0, osem0),
            (el1, er1, sem1, exb1, osem1)]
    iota = jax.lax.iota(jnp.int32, 16)

    def fire(i, buf):
        el_v, er_v, sem = buf[:3]
        pltpu.async_copy(el_ref.at[idx_s.at[i]], el_v, sem)
        pltpu.async_copy(er_ref.at[idx_d.at[i]], er_v, sem)

    def drain(buf):
        el_v, er_v, sem = buf[:3]
        pltpu.make_async_copy(el_ref.at[pl.ds(0, B)], el_v, sem).wait()
        pltpu.make_async_copy(er_ref.at[pl.ds(0, B)], er_v, sem).wait()

    def compute(i, buf):
        el_v, er_v, _, exb, osem = buf
        bi = w * KB1 + i
        m = jnp.where(bi < NBR, jnp.float32(1.0), jnp.float32(0.0))

        @pl.when(i >= 2)
        def _():
            pltpu.make_async_copy(exb, ex_ref.at[pl.ds(0, B)], osem).wait()

        def group_body(g, _):
            ridx = g * 16 + iota

            def dim_body(d, score):
                cidx = jnp.broadcast_to(d, (16,))
                x = (plsc.load_gather(el_v, [ridx, cidx]) +
                     plsc.load_gather(er_v, [ridx, cidx]))
                x = jnp.maximum(x, 0.2 * x)
                return score + attn_sm[d] * x

            score = lax.fori_loop(0, D, dim_body,
                                  jnp.zeros((16,), jnp.float32))
            ex16 = jnp.exp(score) * m
            exb[pl.ds(g * 16, 16)] = ex16
            plsc.store_scatter(ex8_v, [ridx, jnp.zeros((16,), jnp.int32)],
                               ex16)
            return 0

        lax.fori_loop(0, B // 16, group_body, 0)
        pltpu.sync_copy(ex8_v, den_sh.at[idx_d.at[i]], add=True)
        pltpu.async_copy(exb, ex_ref.at[pl.ds(bi * B, B)], osem)

    fire(0, bufs[0])

    def body(g, _):
        i0 = 2 * g
        i1 = 2 * g + 1
        fire(i1, bufs[1])
        drain(bufs[0])
        compute(i0, bufs[0])

        @pl.when(i0 + 2 < KB1)
        def _():
            fire(i0 + 2, bufs[0])

        drain(bufs[1])
        compute(i1, bufs[1])

        @pl.when(i1 + 2 < KB1)
        def _():
            fire(i1 + 2, bufs[1])

        return 0

    lax.fori_loop(0, KB1 // 2, body, 0)
    pltpu.make_async_copy(exb0, ex_ref.at[pl.ds(0, B)], osem0).wait()
    pltpu.make_async_copy(exb1, ex_ref.at[pl.ds(0, B)], osem1).wait()
    plsc.subcore_barrier()

    def flush_body(i, _):
        k = i * NS + sid
        pltpu.sync_copy(den_sh.at[pl.ds(k * FC, FC)],
                        den_ref.at[ci, pl.ds(k * FC, FC)])
        return 0

    lax.fori_loop(0, nzc, flush_body, 0)


@functools.partial(
    pl.kernel,
    mesh=_mesh,
    compiler_params=pltpu.CompilerParams(needs_layout_passes=False,
                                         use_tc_tiling_on_sc=False),
    out_type=[jax.ShapeDtypeStruct((EPAD,), jnp.float32),
              jax.ShapeDtypeStruct((NC, N, 8), jnp.float32)],
    scratch_types=[
        pltpu.VMEM((KB1, B), jnp.int32),
        pltpu.VMEM((KB1, B), jnp.int32),
        pltpu.VMEM((B, D), jnp.float32),
        pltpu.VMEM((B, D), jnp.float32),
        pltpu.VMEM((B, D), jnp.float32),
        pltpu.VMEM((B, D), jnp.float32),
        pltpu.VMEM((B,), jnp.float32),
        pltpu.VMEM((B,), jnp.float32),
        pltpu.VMEM((B, 8), jnp.float32),
        pltpu.VMEM((D,), jnp.float32),
        pltpu.VMEM_SHARED((D,), jnp.float32),
        pltpu.SMEM((D,), jnp.float32),
        pltpu.VMEM_SHARED((N, 8), jnp.float32),
        pltpu.SemaphoreType.DMA,
        pltpu.SemaphoreType.DMA,
        pltpu.SemaphoreType.DMA,
        pltpu.SemaphoreType.DMA,
    ],
)
def _k1(el_ref, er_ref, attn_ref, src_ref, dst_ref,
        zer_ref, ex_ref, den_ref, *scratch):
    _k1_body(el_ref, er_ref, attn_ref,
             src_ref, dst_ref, zer_ref, ex_ref, den_ref, *scratch)


def _k2_body(ell_ref, elh_ref, ex_ref, src_ref, dst_ref, zer_ref, out_ref,
             idx_s, idx_d, rows0, exv0, rows1, exv1, out_sh, sem0, sem1):
    ci = lax.axis_index("c")
    sid = lax.axis_index("s")

    def zero_body(i, _):
        k = i * NS + sid
        pltpu.sync_copy(zer_ref, out_sh.at[pl.ds(k * FC, FC)])
        return 0

    nzc = NFC // NS + jnp.where(sid < NFC % NS, 1, 0)
    lax.fori_loop(0, nzc, zero_body, 0)
    plsc.subcore_barrier()

    bufs = [(rows0, exv0, sem0), (rows1, exv1, sem1)]
    iota = jax.lax.iota(jnp.int32, 16)

    def fire(c, i, buf):
        rows_v, ex_v, sem = buf
        bi = sid * KB2 + c * CH2 + i

        @pl.when(ci == 0)
        def _():
            pltpu.async_copy(ell_ref.at[idx_s.at[i]], rows_v, sem)

        @pl.when(ci == 1)
        def _():
            pltpu.async_copy(elh_ref.at[idx_s.at[i]], rows_v, sem)

        pltpu.async_copy(ex_ref.at[pl.ds(bi * B, B)], ex_v, sem)

    def drain(buf):
        rows_v, ex_v, sem = buf
        pltpu.make_async_copy(ell_ref.at[pl.ds(0, B)], rows_v, sem).wait()
        pltpu.make_async_copy(ex_ref.at[pl.ds(0, B)], ex_v, sem).wait()

    def compute(i, buf):
        rows_v, ex_v, _ = buf

        def group_body(g, _):
            ridx = g * 16 + iota
            ex16 = ex_v[pl.ds(g * 16, 16)]

            def dim_body(d, _):
                cidx = jnp.broadcast_to(d, (16,))
                col = plsc.load_gather(rows_v, [ridx, cidx])
                plsc.store_scatter(rows_v, [ridx, cidx], col * ex16)
                return 0

            lax.fori_loop(0, DH, dim_body, 0)
            return 0

        lax.fori_loop(0, B // 16, group_body, 0)
        pltpu.sync_copy(rows_v, out_sh.at[idx_d.at[i]], add=True)

    def chunk_body(c, _):
        pltpu.sync_copy(src_ref.at[pl.ds(sid * KB2 + c * CH2, CH2)], idx_s)
        pltpu.sync_copy(dst_ref.at[pl.ds(sid * KB2 + c * CH2, CH2)], idx_d)
        fire(c, 0, bufs[0])

        def body(g, _):
            i0 = 2 * g
            i1 = 2 * g + 1
            fire(c, i1, bufs[1])
            drain(bufs[0])
            compute(i0, bufs[0])

            @pl.when(i0 + 2 < CH2)
            def _():
                fire(c, i0 + 2, bufs[0])

            drain(bufs[1])
            compute(i1, bufs[1])

            @pl.when(i1 + 2 < CH2)
            def _():
                fire(c, i1 + 2, bufs[1])

            return 0

        lax.fori_loop(0, CH2 // 2, body, 0)
        return 0

    lax.fori_loop(0, KB2 // CH2, chunk_body, 0)
    plsc.subcore_barrier()

    def flush_body(i, _):
        k = i * NS + sid
        pltpu.sync_copy(out_sh.at[pl.ds(k * FC, FC)],
                        out_ref.at[ci, pl.ds(k * FC, FC)])
        return 0

    lax.fori_loop(0, nzc, flush_body, 0)


@functools.partial(
    pl.kernel,
    mesh=_mesh,
    compiler_params=pltpu.CompilerParams(needs_layout_passes=False,
                                         use_tc_tiling_on_sc=False),
    out_type=jax.ShapeDtypeStruct((NC, N, DH), jnp.float32),
    scratch_types=[
        pltpu.VMEM((CH2, B), jnp.int32),
        pltpu.VMEM((CH2, B), jnp.int32),
        pltpu.VMEM((B, DH), jnp.float32),
        pltpu.VMEM((B,), jnp.float32),
        pltpu.VMEM((B, DH), jnp.float32),
        pltpu.VMEM((B,), jnp.float32),
        pltpu.VMEM_SHARED((N, DH), jnp.float32),
        pltpu.SemaphoreType.DMA,
        pltpu.SemaphoreType.DMA,
    ],
)
def _k2(ell_ref, elh_ref, ex_ref, src_ref, dst_ref, zer_ref, out_ref,
        *scratch):
    _k2_body(ell_ref, elh_ref, ex_ref, src_ref, dst_ref, zer_ref, out_ref,
             *scratch)


def _a1_body(u_ref, i_ref, outr_ref, outb_ref, outs_ref,
             denr_ref, denb_ref, dens_ref,
             w1f_ref, b1f_ref, w2f_ref, b2f_ref,
             w1t_ref, b1t_ref, w2t_ref, b2t_ref,
             inew_ref, p_ref, q_ref, sinf_ref, sint_ref, mom_ref):
    step = pl.program_id(0)

    def seg_norm(o_ref, den_ref):
        den = den_ref[0, :, 0] + den_ref[1, :, 0] + 1e-9
        num = jnp.concatenate([o_ref[0, :, :], o_ref[1, :, :]], axis=1)
        return num / den[:, None]

    item_agg = seg_norm(outr_ref, denr_ref)
    inew_ref[...] = item_agg + i_ref[...]
    q = seg_norm(outb_ref, denb_ref)
    p = seg_norm(outs_ref, dens_ref)
    p_ref[...] = p
    q_ref[...] = q

    u = u_ref[...]

    def att_s(xr, w1_ref, b1_ref, w2_ref, b2_ref):
        x = jnp.concatenate([u, xr], axis=1)
        h = lax.dot_general(x, w1_ref[...], (((1,), (0,)), ((), ())),
                            preferred_element_type=jnp.float32) + b1_ref[...]
        s = lax.dot_general(h, w2_ref[...], (((1,), (0,)), ((), ())),
                            preferred_element_type=jnp.float32) + b2_ref[...]
        return s

    s_inf = att_s(p, w1f_ref, b1f_ref, w2f_ref, b2f_ref)
    s_int = att_s(q, w1t_ref, b1t_ref, w2t_ref, b2t_ref)
    sinf_ref[...] = s_inf
    sint_ref[...] = s_int

    @pl.when(step == 0)
    def _():
        mom_ref[...] = jnp.zeros((1, 8), jnp.float32)

    upd = jnp.concatenate([
        jnp.sum(s_inf)[None], jnp.sum(s_inf * s_inf)[None],
        jnp.sum(s_int)[None], jnp.sum(s_int * s_int)[None],
        jnp.zeros((4,), jnp.float32)]).reshape(1, 8)
    mom_ref[...] = mom_ref[...] + upd


def _a1(emb_u, emb_i, outr, outb, outs, denr, denb, dens, att_inf, att_int):
    n_blocks = N // _R
    emb_spec = pl.BlockSpec((_R, D), lambda i: (i, 0))
    o_spec = pl.BlockSpec((2, _R, DH), lambda i: (0, i, 0))
    den_spec = pl.BlockSpec((2, _R, 8), lambda i: (0, i, 0))
    w1_spec = pl.BlockSpec((2 * D, 2 * D), lambda i: (0, 0))
    b1_spec = pl.BlockSpec((1, 2 * D), lambda i: (0, 0))
    w2_spec = pl.BlockSpec((2 * D, 1), lambda i: (0, 0))
    b2_spec = pl.BlockSpec((1, 1), lambda i: (0, 0))
    s_spec = pl.BlockSpec((_R, 1), lambda i: (i, 0))
    mom_spec = pl.BlockSpec((1, 8), lambda i: (0, 0))
    return pl.pallas_call(
        _a1_body,
        grid=(n_blocks,),
        in_specs=[emb_spec, emb_spec, o_spec, o_spec, o_spec,
                  den_spec, den_spec, den_spec,
                  w1_spec, b1_spec, w2_spec, b2_spec,
                  w1_spec, b1_spec, w2_spec, b2_spec],
        out_specs=[emb_spec, emb_spec, emb_spec, s_spec, s_spec, mom_spec],
        out_shape=[jax.ShapeDtypeStruct((N, D), jnp.float32),
                   jax.ShapeDtypeStruct((N, D), jnp.float32),
                   jax.ShapeDtypeStruct((N, D), jnp.float32),
                   jax.ShapeDtypeStruct((N, 1), jnp.float32),
                   jax.ShapeDtypeStruct((N, 1), jnp.float32),
                   jax.ShapeDtypeStruct((1, 8), jnp.float32)],
    )(emb_u, emb_i, outr, outb, outs, denr, denb, dens,
      att_inf["W1"], att_inf["b1"].reshape(1, 2 * D),
      att_inf["W2"], att_inf["b2"].reshape(1, 1),
      att_int["W1"], att_int["b1"].reshape(1, 2 * D),
      att_int["W2"], att_int["b2"].reshape(1, 1))


def _a2_body(p_ref, q_ref, u_ref, sinf_ref, sint_ref, mom_ref, unew_ref):
    n_f = jnp.float32(N)
    mu_f = mom_ref[0, 0] / n_f
    var_f = mom_ref[0, 1] / n_f - mu_f * mu_f
    mu_t = mom_ref[0, 2] / n_f
    var_t = mom_ref[0, 3] / n_f - mu_t * mu_t
    a = (sinf_ref[...] - mu_f) * lax.rsqrt(var_f + 1e-5)
    b = (sint_ref[...] - mu_t) * lax.rsqrt(var_t + 1e-5)
    a = jnp.maximum(a, 0.01 * a)
    b = jnp.maximum(b, 0.01 * b)
    m = jnp.maximum(a, b)
    ea = jnp.exp(a - m)
    eb = jnp.exp(b - m)
    tot = ea + eb
    g0 = ea / tot
    g1 = eb / tot
    unew_ref[...] = g0 * p_ref[...] + g1 * q_ref[...] + u_ref[...]


def _a2(p_hair, q_hair, emb_u, s_inf, s_int, mom):
    n_blocks = N // _R
    emb_spec = pl.BlockSpec((_R, D), lambda i: (i, 0))
    s_spec = pl.BlockSpec((_R, 1), lambda i: (i, 0))
    mom_spec = pl.BlockSpec((1, 8), lambda i: (0, 0))
    return pl.pallas_call(
        _a2_body,
        grid=(n_blocks,),
        in_specs=[emb_spec, emb_spec, emb_spec, s_spec, s_spec, mom_spec],
        out_specs=emb_spec,
        out_shape=jax.ShapeDtypeStruct((N, D), jnp.float32),
    )(p_hair, q_hair, emb_u, s_inf, s_int, mom)


def _pred_body(ru_ref, ri_ref, iu_ref, ii_ref, out_ref,
               idx_u, idx_i, ru0, ri0, ru1, ri1, sc_v, sem0, sem1):
    ci = lax.axis_index("c")
    sid = lax.axis_index("s")
    w = sid * NC + ci
    iota = jax.lax.iota(jnp.int32, 16)

    pltpu.sync_copy(iu_ref.at[pl.ds(w * KBP, KBP)], idx_u)
    pltpu.sync_copy(ii_ref.at[pl.ds(w * KBP, KBP)], idx_i)

    bufs = [(ru0, ri0, sem0), (ru1, ri1, sem1)]

    def fire(i, buf):
        ru_v, ri_v, sem = buf
        pltpu.async_copy(ru_ref.at[idx_u.at[i]], ru_v, sem)
        pltpu.async_copy(ri_ref.at[idx_i.at[i]], ri_v, sem)

    def drain(buf):
        ru_v, ri_v, sem = buf
        pltpu.make_async_copy(ru_ref.at[pl.ds(0, PB)], ru_v, sem).wait()
        pltpu.make_async_copy(ri_ref.at[pl.ds(0, PB)], ri_v, sem).wait()

    def compute(i, buf):
        ru_v, ri_v, _ = buf

        def group_body(g, _):
            ridx = g * 16 + iota

            def dim_body(d, acc):
                cidx = jnp.broadcast_to(d, (16,))
                uu = plsc.load_gather(ru_v, [ridx, cidx])
                vv = plsc.load_gather(ri_v, [ridx, cidx])
                return acc + uu * vv

            acc = lax.fori_loop(0, RD, dim_body, jnp.zeros((16,), jnp.float32))
            sc_v[pl.ds(i * PB + g * 16, 16)] = acc
            return 0

        lax.fori_loop(0, PB // 16, group_body, 0)

    fire(0, bufs[0])

    def body(g, _):
        i0 = 2 * g
        i1 = 2 * g + 1
        fire(i1, bufs[1])
        drain(bufs[0])
        compute(i0, bufs[0])

        @pl.when(i0 + 2 < KBP)
        def _():
            fire(i0 + 2, bufs[0])

        drain(bufs[1])
        compute(i1, bufs[1])

        @pl.when(i1 + 2 < KBP)
        def _():
            fire(i1 + 2, bufs[1])

        return 0

    lax.fori_loop(0, KBP // 2, body, 0)
    pltpu.sync_copy(sc_v, out_ref.at[pl.ds(w * KBP * PB, KBP * PB)])


@functools.partial(
    pl.kernel,
    mesh=_mesh,
    compiler_params=pltpu.CompilerParams(needs_layout_passes=False,
                                         use_tc_tiling_on_sc=False),
    out_type=jax.ShapeDtypeStruct((PPAD,), jnp.float32),
    scratch_types=[
        pltpu.VMEM((KBP, PB), jnp.int32),
        pltpu.VMEM((KBP, PB), jnp.int32),
        pltpu.VMEM((PB, RD), jnp.float32),
        pltpu.VMEM((PB, RD), jnp.float32),
        pltpu.VMEM((PB, RD), jnp.float32),
        pltpu.VMEM((PB, RD), jnp.float32),
        pltpu.VMEM((KBP * PB,), jnp.float32),
        pltpu.SemaphoreType.DMA,
        pltpu.SemaphoreType.DMA,
    ],
)
def _pred(ru_ref, ri_ref, iu_ref, ii_ref, *scratch):
    _pred_body(ru_ref, ri_ref, iu_ref, ii_ref, *scratch)


def _pad2d(x, nb, b):
    x = x.astype(jnp.int32)
    pad = nb * b - x.shape[0]
    return jnp.concatenate([x, jnp.zeros((pad,), jnp.int32)]).reshape(nb, b)


def kernel(ui_edge_index, uu_edge_index, pos_edge_index, neg_edge_index, params):
    ui0 = _pad2d(ui_edge_index[0], NBT, B)
    ui1 = _pad2d(ui_edge_index[1], NBT, B)
    uu0 = _pad2d(uu_edge_index[0], NBT, B)
    uu1 = _pad2d(uu_edge_index[1], NBT, B)

    zer8 = jnp.zeros((FC, 8), jnp.float32)
    zer32 = jnp.zeros((FC, DH), jnp.float32)

    emb_u = params["user_emb"]
    emb_i = params["item_emb"]
    res_u = [emb_u]
    res_i = [emb_i]

    for lp in params["layers"]:
        (el_r, el64_r, er64_r, el_b, el64_b, er64_b,
         el_s, el64_s, er64_s) = _proj(emb_u, emb_i, lp)

        ex_r, den_r = _k1(el64_r, er64_r, lp["rate"]["attn"], ui0, ui1, zer8)
        ex_b, den_b = _k1(el64_b, er64_b, lp["rated_by"]["attn"],
                          ui1, ui0, zer8)
        ex_s, den_s = _k1(el64_s, er64_s, lp["social"]["attn"],
                          uu0, uu1, zer8)

        outr = _k2(el_r[0], el_r[1], ex_r, ui0, ui1, zer32)
        outb = _k2(el_b[0], el_b[1], ex_b, ui1, ui0, zer32)
        outs = _k2(el_s[0], el_s[1], ex_s, uu0, uu1, zer32)

        emb_i_new, p_hair, q_hair, s_inf, s_int, mom = _a1(
            emb_u, emb_i, outr, outb, outs, den_r, den_b, den_s,
            lp["att_inf"], lp["att_int"])
        emb_u = _a2(p_hair, q_hair, emb_u, s_inf, s_int, mom)
        emb_i = emb_i_new
        res_u.append(emb_u)
        res_i.append(emb_i)

    res_u_cat = jnp.concatenate(res_u, axis=1)
    res_i_cat = jnp.concatenate(res_i, axis=1)

    iu = _pad2d(jnp.concatenate([pos_edge_index[0], neg_edge_index[0]]),
                PNBT, PB)
    ii = _pad2d(jnp.concatenate([pos_edge_index[1], neg_edge_index[1]]),
                PNBT, PB)
    scores = _pred(res_u_cat, res_i_cat, iu, ii)
    pos_score = scores[:EP, None]
    neg_score = scores[EP:2 * EP, None]
    return pos_score, neg_score
